# Initial kernel scaffold; baseline (speedup 1.0000x reference)
import jax
import jax.numpy as jnp
from jax import lax
from jax.experimental import pallas as pl
from jax.experimental.pallas import tpu as pltpu
from jax.experimental.pallas import tpu_sc as plsc

_B = 512
_NC = 2
_NS = 16
_NW = _NC * _NS
_C = 80
_L = 16


def _logits_body(x_ref, w1_ref, b1_ref, w2_ref, b2_ref, e_ref):
    t = jnp.tanh(
        jnp.dot(x_ref[...], w1_ref[...], preferred_element_type=jnp.float32)
        + b1_ref[...]
    )
    logit = jnp.sum(t * w2_ref[...], axis=1, keepdims=True) + b2_ref[...]
    e_ref[...] = jnp.exp(logit)


def _pool_body(x_hbm, e_hbm, idx_hbm, outp_hbm, outd_hbm, xb, eb, ib, db, acc, den):
    n = x_hbm.shape[0]
    d = xb.shape[1]
    cid = lax.axis_index("c")
    sid = lax.axis_index("s")
    wid = sid * _NC + cid

    zero = jnp.zeros((_L,), jnp.float32)
    rows_per_tile = _B // _NS

    def _zrow(r, carry):
        def _zcol(j, c2):
            xb[r, pl.ds(j * _L, _L)] = zero
            return c2
        lax.fori_loop(0, d // _L, _zcol, 0)
        db[r, :] = zero
        return carry

    lax.fori_loop(0, rows_per_tile, _zrow, 0)
    seg0 = sid * rows_per_tile
    pltpu.sync_copy(xb.at[pl.ds(0, rows_per_tile)], acc.at[pl.ds(seg0, rows_per_tile)])
    pltpu.sync_copy(db.at[pl.ds(0, rows_per_tile)], den.at[pl.ds(seg0, rows_per_tile)])
    plsc.subcore_barrier()

    n_chunks = n // _C
    base = n_chunks // _NW
    extra = n_chunks - base * _NW
    n_t = base + jnp.where(wid < extra, 1, 0)

    def _chunk(t, carry):
        r0 = (wid + t * _NW) * _C
        pltpu.sync_copy(x_hbm.at[pl.ds(r0, _C)], xb)
        pltpu.sync_copy(e_hbm.at[pl.ds(r0, _C)], eb)
        pltpu.sync_copy(idx_hbm.at[pl.ds(r0, _C)], ib)

        def _row(r, c2):
            ev = eb[r]
            for j in range(d // _L):
                xb[r, pl.ds(j * _L, _L)] = xb[r, pl.ds(j * _L, _L)] * ev
            db[r, :] = jnp.full((_L,), ev, jnp.float32)
            return c2

        lax.fori_loop(0, _C, _row, 0)
        pltpu.sync_copy(xb, acc.at[ib], add=True)
        pltpu.sync_copy(db, den.at[ib], add=True)
        return carry

    lax.fori_loop(0, n_t, _chunk, 0)
    plsc.subcore_barrier()

    pltpu.sync_copy(
        acc.at[pl.ds(seg0, rows_per_tile)],
        outp_hbm.at[cid, pl.ds(seg0, rows_per_tile)],
    )
    pltpu.sync_copy(
        den.at[pl.ds(seg0, rows_per_tile)],
        outd_hbm.at[cid, pl.ds(seg0, rows_per_tile)],
    )


def _combine_body(p_ref, d_ref, o_ref):
    num = p_ref[0] + p_ref[1]
    dsum = d_ref[0][:, 0:1] + d_ref[1][:, 0:1]
    o_ref[...] = num / (dsum + 1e-16)


def kernel(x, batch, W1, b1, W2, b2):
    n, d = x.shape
    h = W1.shape[1]
    r = 2000
    e = pl.pallas_call(
        _logits_body,
        grid=(n // r,),
        in_specs=[
            pl.BlockSpec((r, d), lambda i: (i, 0)),
            pl.BlockSpec((d, h), lambda i: (0, 0)),
            pl.BlockSpec((1, h), lambda i: (0, 0)),
            pl.BlockSpec((1, h), lambda i: (0, 0)),
            pl.BlockSpec((1, 1), lambda i: (0, 0)),
        ],
        out_specs=pl.BlockSpec((r, 1), lambda i: (i, 0)),
        out_shape=jax.ShapeDtypeStruct((n, 1), jnp.float32),
    )(x, W1, b1.reshape(1, h), W2.reshape(1, h), b2.reshape(1, 1))

    mesh = plsc.VectorSubcoreMesh(
        core_axis_name="c", subcore_axis_name="s", num_cores=_NC, num_subcores=_NS
    )
    pool = pl.kernel(
        _pool_body,
        out_type=[
            jax.ShapeDtypeStruct((_NC, _B, d), jnp.float32),
            jax.ShapeDtypeStruct((_NC, _B, _L), jnp.float32),
        ],
        mesh=mesh,
        scratch_types=[
            pltpu.VMEM((_C, d), jnp.float32),
            pltpu.VMEM((_C,), jnp.float32),
            pltpu.VMEM((_C,), jnp.int32),
            pltpu.VMEM((_C, _L), jnp.float32),
            pltpu.VMEM_SHARED((_B, d), jnp.float32),
            pltpu.VMEM_SHARED((_B, _L), jnp.float32),
        ],
    )
    partials, dens = pool(x, e.reshape(n), batch)

    out = pl.pallas_call(
        _combine_body,
        out_shape=jax.ShapeDtypeStruct((_B, d), jnp.float32),
    )(partials, dens)
    return out


# speedup vs baseline: 4.1548x; 4.1548x over previous
import jax
import jax.numpy as jnp
from jax import lax
from jax.experimental import pallas as pl
from jax.experimental.pallas import tpu as pltpu
from jax.experimental.pallas import tpu_sc as plsc

_B = 512
_NC = 2
_NS = 16
_NW = _NC * _NS
_CB = 4
_RG = _NW // _CB
_W = 128
_C = 160
_L = 16


def _logits_body(x_ref, w1_ref, b1_ref, w2_ref, b2_ref, e_ref):
    t = jnp.tanh(
        jnp.dot(x_ref[...], w1_ref[...], preferred_element_type=jnp.float32)
        + b1_ref[...]
    )
    logit = jnp.sum(t * w2_ref[...], axis=1, keepdims=True) + b2_ref[...]
    e_ref[...] = jnp.exp(logit)


def _tree_sum(vals):
    while len(vals) > 1:
        nxt = [vals[i] + vals[i + 1] for i in range(0, len(vals) - 1, 2)]
        if len(vals) % 2:
            nxt.append(vals[-1])
        vals = nxt
    return vals[0]


_NV = _W // _L


def _pool_body(
    x_hbm, e_hbm, idx_hbm, outp_hbm, outd_hbm, xb, eb, ib, acc, dacc, curbuf, curden
):
    n = x_hbm.shape[0]
    cid = lax.axis_index("c")
    sid = lax.axis_index("s")
    wid = sid * _NC + cid
    cb = wid % _CB
    rg = wid // _CB
    col0 = cb * _W

    zero = jnp.zeros((_L,), jnp.float32)

    def _zr(r, carry):
        for j in range(_NV):
            acc[r, pl.ds(j * _L, _L)] = zero
        dacc[pl.ds(r * _L, _L)] = zero
        return carry

    lax.fori_loop(0, _B + 1, _zr, 0)
    for j in range(_NV):
        curbuf[pl.ds(j * _L, _L)] = zero
    curden[...] = zero

    n_chunks = n // _C
    base_ct = n_chunks // _RG
    n_t = base_ct + jnp.where(rg < n_chunks - base_ct * _RG, 1, 0)

    def _flush(seg):
        for j in range(_NV):
            sl = pl.ds(j * _L, _L)
            plsc.addupdate(acc.at[seg, sl], curbuf[sl])
            curbuf[sl] = zero
        plsc.addupdate(dacc.at[pl.ds(seg * _L, _L)], curden[...])
        curden[...] = zero

    def _chunk(t, carry):
        r0 = (rg + t * _RG) * _C
        pltpu.sync_copy(x_hbm.at[pl.ds(r0, _C), pl.ds(col0, _W)], xb)
        pltpu.sync_copy(e_hbm.at[pl.ds(r0, _C)], eb)
        pltpu.sync_copy(idx_hbm.at[pl.ds(r0, _C)], ib)

        def _grp(g, cur_seg):
            rbase = g * _L
            e16 = eb[pl.ds(rbase, _L)]
            s16 = ib[pl.ds(rbase, _L)]
            first = s16[0]
            last = s16[_L - 1]
            hetero = first != last

            @pl.when(hetero | (first != cur_seg))
            def _():
                _flush(cur_seg)

            @pl.when(jnp.logical_not(hetero))
            def _():
                ev = [jnp.full((_L,), e16[k], jnp.float32) for k in range(_L)]
                for j in range(_NV):
                    sl = pl.ds(j * _L, _L)
                    contrib = _tree_sum(
                        [xb[rbase + k, sl] * ev[k] for k in range(_L)]
                    )
                    plsc.addupdate(curbuf.at[sl], contrib)
                plsc.addupdate(curden.at[pl.ds(0, _L)], e16)

            @pl.when(hetero)
            def _():
                for k in range(_L):
                    ev = jnp.full((_L,), e16[k], jnp.float32)
                    for j in range(_NV):
                        sl = pl.ds(j * _L, _L)
                        plsc.addupdate(acc.at[s16[k], sl], xb[rbase + k, sl] * ev)
                    plsc.addupdate(dacc.at[pl.ds(s16[k] * _L, _L)], ev * (1.0 / _L))

            return last

        return lax.fori_loop(0, _C // _L, _grp, carry)

    cur_seg = lax.fori_loop(0, n_t, _chunk, jnp.int32(_B))
    _flush(cur_seg)

    pltpu.sync_copy(acc.at[pl.ds(0, _B)], outp_hbm.at[rg, :, pl.ds(col0, _W)])

    @pl.when(cb == 0)
    def _():
        pltpu.sync_copy(dacc.at[pl.ds(0, _B * _L)], outd_hbm.at[rg])


def _combine_body(p_ref, d_ref, o_ref):
    num = jnp.sum(p_ref[...], axis=0)
    dsum = jnp.sum(d_ref[...], axis=(0, 2))[:, None]
    o_ref[...] = num / (dsum + 1e-16)


def kernel(x, batch, W1, b1, W2, b2):
    n, d = x.shape
    h = W1.shape[1]
    r = 2000
    e = pl.pallas_call(
        _logits_body,
        grid=(n // r,),
        in_specs=[
            pl.BlockSpec((r, d), lambda i: (i, 0)),
            pl.BlockSpec((d, h), lambda i: (0, 0)),
            pl.BlockSpec((1, h), lambda i: (0, 0)),
            pl.BlockSpec((1, h), lambda i: (0, 0)),
            pl.BlockSpec((1, 1), lambda i: (0, 0)),
        ],
        out_specs=pl.BlockSpec((r, 1), lambda i: (i, 0)),
        out_shape=jax.ShapeDtypeStruct((n, 1), jnp.float32),
    )(x, W1, b1.reshape(1, h), W2.reshape(1, h), b2.reshape(1, 1))

    mesh = plsc.VectorSubcoreMesh(
        core_axis_name="c", subcore_axis_name="s", num_cores=_NC, num_subcores=_NS
    )
    pool = pl.kernel(
        _pool_body,
        out_type=[
            jax.ShapeDtypeStruct((_RG, _B, d), jnp.float32),
            jax.ShapeDtypeStruct((_RG, _B * _L), jnp.float32),
        ],
        mesh=mesh,
        scratch_types=[
            pltpu.VMEM((_C, _W), jnp.float32),
            pltpu.VMEM((_C,), jnp.float32),
            pltpu.VMEM((_C,), jnp.int32),
            pltpu.VMEM((_B + 1, _W), jnp.float32),
            pltpu.VMEM(((_B + 1) * _L,), jnp.float32),
            pltpu.VMEM((_W,), jnp.float32),
            pltpu.VMEM((_L,), jnp.float32),
        ],
    )
    partials, dens = pool(x, e.reshape(n), batch)

    out = pl.pallas_call(
        _combine_body,
        out_shape=jax.ShapeDtypeStruct((_B, d), jnp.float32),
    )(partials, dens.reshape(_RG, _B, _L))
    return out


# speedup vs baseline: 6.4298x; 1.5476x over previous
import jax
import jax.numpy as jnp
from jax import lax
from jax.experimental import pallas as pl
from jax.experimental.pallas import tpu as pltpu
from jax.experimental.pallas import tpu_sc as plsc

_B = 512
_NC = 2
_NS = 16
_NW = _NC * _NS
_CB = 4
_RG = _NW // _CB
_W = 128
_C = 160
_L = 16


def _logits_body(x_ref, w1_ref, b1_ref, w2_ref, b2_ref, e_ref):
    t = jnp.tanh(
        jnp.dot(x_ref[...], w1_ref[...], preferred_element_type=jnp.float32)
        + b1_ref[...]
    )
    logit = jnp.sum(t * w2_ref[...], axis=1, keepdims=True) + b2_ref[...]
    e_ref[...] = jnp.exp(logit)


def _tree_sum(vals):
    while len(vals) > 1:
        nxt = [vals[i] + vals[i + 1] for i in range(0, len(vals) - 1, 2)]
        if len(vals) % 2:
            nxt.append(vals[-1])
        vals = nxt
    return vals[0]


_NV = _W // _L


def _pool_body(
    x_hbm, e_hbm, idx_hbm, outp_hbm, outd_hbm,
    xb0, eb0, ib0, xb1, eb1, ib1, acc, dacc, curbuf, curden, segref,
    sem0, sem1,
):
    n = x_hbm.shape[0]
    cid = lax.axis_index("c")
    sid = lax.axis_index("s")
    wid = sid * _NC + cid
    cb = wid % _CB
    rg = wid // _CB
    col0 = cb * _W

    zero = jnp.zeros((_L,), jnp.float32)
    bufs = ((xb0, eb0, ib0, sem0), (xb1, eb1, ib1, sem1))

    def _zr(r, carry):
        for j in range(_NV):
            acc[r, pl.ds(j * _L, _L)] = zero
        dacc[pl.ds(r * _L, _L)] = zero
        return carry

    lax.fori_loop(0, _B + 1, _zr, 0)
    for j in range(_NV):
        curbuf[pl.ds(j * _L, _L)] = zero
    curden[...] = zero
    segref[0] = jnp.int32(_B)

    n_chunks = n // _C
    base_ct = n_chunks // _RG
    n_t = base_ct + jnp.where(rg < n_chunks - base_ct * _RG, 1, 0)

    def _flush(seg):
        for j in range(_NV):
            sl = pl.ds(j * _L, _L)
            plsc.addupdate(acc.at[seg, sl], curbuf[sl])
            curbuf[sl] = zero
        plsc.addupdate(dacc.at[pl.ds(seg * _L, _L)], curden[...])
        curden[...] = zero

    def _start(buf, t):
        xb, eb, ib, sem = buf
        r0 = (rg + t * _RG) * _C
        pltpu.make_async_copy(
            x_hbm.at[pl.ds(r0, _C), pl.ds(col0, _W)], xb, sem
        ).start()
        pltpu.make_async_copy(e_hbm.at[pl.ds(r0, _C)], eb, sem).start()
        pltpu.make_async_copy(idx_hbm.at[pl.ds(r0, _C)], ib, sem).start()

    def _wait(buf):
        xb, eb, ib, sem = buf
        pltpu.make_async_copy(x_hbm.at[pl.ds(0, _C), pl.ds(0, _W)], xb, sem).wait()
        pltpu.make_async_copy(e_hbm.at[pl.ds(0, _C)], eb, sem).wait()
        pltpu.make_async_copy(idx_hbm.at[pl.ds(0, _C)], ib, sem).wait()

    def _process(buf):
        xb, eb, ib, _ = buf

        def _grp(g, carry):
            cur_seg = segref[0]
            rbase = g * _L
            e16 = eb[pl.ds(rbase, _L)]
            s16 = ib[pl.ds(rbase, _L)]
            first = s16[0]
            last = s16[_L - 1]
            hetero = first != last

            @pl.when(hetero | (first != cur_seg))
            def _():
                _flush(cur_seg)

            @pl.when(jnp.logical_not(hetero))
            def _():
                ev = [jnp.full((_L,), e16[k], jnp.float32) for k in range(_L)]
                for j in range(_NV):
                    sl = pl.ds(j * _L, _L)
                    contrib = _tree_sum(
                        [xb[rbase + k, sl] * ev[k] for k in range(_L)]
                    )
                    plsc.addupdate(curbuf.at[sl], contrib)
                plsc.addupdate(curden.at[pl.ds(0, _L)], e16)

            @pl.when(hetero)
            def _():
                for k in range(_L):
                    ev = jnp.full((_L,), e16[k], jnp.float32)
                    for j in range(_NV):
                        sl = pl.ds(j * _L, _L)
                        plsc.addupdate(acc.at[s16[k], sl], xb[rbase + k, sl] * ev)
                    plsc.addupdate(dacc.at[pl.ds(s16[k] * _L, _L)], ev * (1.0 / _L))

            segref[0] = last
            return carry

        lax.fori_loop(0, _C // _L, _grp, 0)

    _start(bufs[0], 0)

    def _chunk2(t2, carry):
        for p in range(2):
            t = t2 * 2 + p

            @pl.when(t + 1 < n_t)
            def _():
                _start(bufs[1 - p], t + 1)

            @pl.when(t < n_t)
            def _():
                _wait(bufs[p])
                _process(bufs[p])
        return carry

    lax.fori_loop(0, (base_ct + 2) // 2, _chunk2, 0)
    _flush(segref[0])

    pltpu.sync_copy(acc.at[pl.ds(0, _B)], outp_hbm.at[rg, :, pl.ds(col0, _W)])

    @pl.when(cb == 0)
    def _():
        pltpu.sync_copy(dacc.at[pl.ds(0, _B * _L)], outd_hbm.at[rg])


def _combine_body(p_ref, d_ref, o_ref):
    num = jnp.sum(p_ref[...], axis=0)
    dsum = jnp.sum(d_ref[...], axis=(0, 2))[:, None]
    o_ref[...] = num / (dsum + 1e-16)


def kernel(x, batch, W1, b1, W2, b2):
    n, d = x.shape
    h = W1.shape[1]
    r = 2000
    e = pl.pallas_call(
        _logits_body,
        grid=(n // r,),
        in_specs=[
            pl.BlockSpec((r, d), lambda i: (i, 0)),
            pl.BlockSpec((d, h), lambda i: (0, 0)),
            pl.BlockSpec((1, h), lambda i: (0, 0)),
            pl.BlockSpec((1, h), lambda i: (0, 0)),
            pl.BlockSpec((1, 1), lambda i: (0, 0)),
        ],
        out_specs=pl.BlockSpec((r, 1), lambda i: (i, 0)),
        out_shape=jax.ShapeDtypeStruct((n, 1), jnp.float32),
    )(x, W1, b1.reshape(1, h), W2.reshape(1, h), b2.reshape(1, 1))

    mesh = plsc.VectorSubcoreMesh(
        core_axis_name="c", subcore_axis_name="s", num_cores=_NC, num_subcores=_NS
    )
    pool = pl.kernel(
        _pool_body,
        out_type=[
            jax.ShapeDtypeStruct((_RG, _B, d), jnp.float32),
            jax.ShapeDtypeStruct((_RG, _B * _L), jnp.float32),
        ],
        mesh=mesh,
        scratch_types=[
            pltpu.VMEM((_C, _W), jnp.float32),
            pltpu.VMEM((_C,), jnp.float32),
            pltpu.VMEM((_C,), jnp.int32),
            pltpu.VMEM((_C, _W), jnp.float32),
            pltpu.VMEM((_C,), jnp.float32),
            pltpu.VMEM((_C,), jnp.int32),
            pltpu.VMEM((_B + 1, _W), jnp.float32),
            pltpu.VMEM(((_B + 1) * _L,), jnp.float32),
            pltpu.VMEM((_W,), jnp.float32),
            pltpu.VMEM((_L,), jnp.float32),
            pltpu.SMEM((1,), jnp.int32),
            pltpu.SemaphoreType.DMA,
            pltpu.SemaphoreType.DMA,
        ],
    )
    partials, dens = pool(x, e.reshape(n), batch)

    out = pl.pallas_call(
        _combine_body,
        out_shape=jax.ShapeDtypeStruct((_B, d), jnp.float32),
    )(partials, dens.reshape(_RG, _B, _L))
    return out


# speedup vs baseline: 6.9074x; 1.0743x over previous
import jax
import jax.numpy as jnp
from jax import lax
from jax.experimental import pallas as pl
from jax.experimental.pallas import tpu as pltpu
from jax.experimental.pallas import tpu_sc as plsc

_B = 512
_NC = 2
_NS = 16
_NW = _NC * _NS
_CB = 4
_RG = _NW // _CB
_W = 128
_C = 160
_L = 16


def _logits_body(x_ref, w1_ref, b1_ref, w2_ref, b2_ref, e_ref):
    t = jnp.tanh(
        jnp.dot(x_ref[...], w1_ref[...], preferred_element_type=jnp.float32)
        + b1_ref[...]
    )
    logit = jnp.sum(t * w2_ref[...], axis=1, keepdims=True) + b2_ref[...]
    e_ref[...] = jnp.exp(logit)


def _tree_sum(vals):
    while len(vals) > 1:
        nxt = [vals[i] + vals[i + 1] for i in range(0, len(vals) - 1, 2)]
        if len(vals) % 2:
            nxt.append(vals[-1])
        vals = nxt
    return vals[0]


_NV = _W // _L


def _pool_body(
    row_lo, nrows,
    x_hbm, e_hbm, idx_hbm, outp_hbm, outd_hbm,
    xb0, eb0, ib0, xb1, eb1, ib1, acc, dacc, curbuf, curden, segref,
    sem0, sem1,
):
    n = nrows
    cid = lax.axis_index("c")
    sid = lax.axis_index("s")
    wid = sid * _NC + cid
    cb = wid % _CB
    rg = wid // _CB
    col0 = cb * _W

    zero = jnp.zeros((_L,), jnp.float32)
    bufs = ((xb0, eb0, ib0, sem0), (xb1, eb1, ib1, sem1))

    def _zr(r, carry):
        for j in range(_NV):
            acc[r, pl.ds(j * _L, _L)] = zero
        dacc[pl.ds(r * _L, _L)] = zero
        return carry

    lax.fori_loop(0, _B + 1, _zr, 0)
    for j in range(_NV):
        curbuf[pl.ds(j * _L, _L)] = zero
    curden[...] = zero
    segref[0] = jnp.int32(_B)

    n_chunks = n // _C
    base_ct = n_chunks // _RG
    n_t = base_ct + jnp.where(rg < n_chunks - base_ct * _RG, 1, 0)

    def _flush(seg):
        for j in range(_NV):
            sl = pl.ds(j * _L, _L)
            plsc.addupdate(acc.at[seg, sl], curbuf[sl])
            curbuf[sl] = zero
        plsc.addupdate(dacc.at[pl.ds(seg * _L, _L)], curden[...])
        curden[...] = zero

    def _start(buf, t):
        xb, eb, ib, sem = buf
        r0 = row_lo + (rg + t * _RG) * _C
        pltpu.make_async_copy(
            x_hbm.at[pl.ds(r0, _C), pl.ds(col0, _W)], xb, sem
        ).start()
        pltpu.make_async_copy(e_hbm.at[pl.ds(r0 - row_lo, _C)], eb, sem).start()
        pltpu.make_async_copy(idx_hbm.at[pl.ds(r0, _C)], ib, sem).start()

    def _wait(buf):
        xb, eb, ib, sem = buf
        pltpu.make_async_copy(x_hbm.at[pl.ds(0, _C), pl.ds(0, _W)], xb, sem).wait()
        pltpu.make_async_copy(e_hbm.at[pl.ds(0, _C)], eb, sem).wait()
        pltpu.make_async_copy(idx_hbm.at[pl.ds(0, _C)], ib, sem).wait()

    def _process(buf):
        xb, eb, ib, _ = buf

        def _grp(g, carry):
            cur_seg = segref[0]
            rbase = g * _L
            e16 = eb[pl.ds(rbase, _L)]
            s16 = ib[pl.ds(rbase, _L)]
            first = s16[0]
            last = s16[_L - 1]
            hetero = first != last

            @pl.when(hetero | (first != cur_seg))
            def _():
                _flush(cur_seg)

            @pl.when(jnp.logical_not(hetero))
            def _():
                ev = [jnp.full((_L,), e16[k], jnp.float32) for k in range(_L)]
                for j in range(_NV):
                    sl = pl.ds(j * _L, _L)
                    contrib = _tree_sum(
                        [xb[rbase + k, sl] * ev[k] for k in range(_L)]
                    )
                    plsc.addupdate(curbuf.at[sl], contrib)
                plsc.addupdate(curden.at[pl.ds(0, _L)], e16)

            @pl.when(hetero)
            def _():
                for k in range(_L):
                    ev = jnp.full((_L,), e16[k], jnp.float32)
                    for j in range(_NV):
                        sl = pl.ds(j * _L, _L)
                        plsc.addupdate(acc.at[s16[k], sl], xb[rbase + k, sl] * ev)
                    plsc.addupdate(dacc.at[pl.ds(s16[k] * _L, _L)], ev * (1.0 / _L))

            segref[0] = last
            return carry

        lax.fori_loop(0, _C // _L, _grp, 0)

    _start(bufs[0], 0)

    def _chunk2(t2, carry):
        for p in range(2):
            t = t2 * 2 + p

            @pl.when(t + 1 < n_t)
            def _():
                _start(bufs[1 - p], t + 1)

            @pl.when(t < n_t)
            def _():
                _wait(bufs[p])
                _process(bufs[p])
        return carry

    lax.fori_loop(0, (base_ct + 2) // 2, _chunk2, 0)
    _flush(segref[0])

    pltpu.sync_copy(acc.at[pl.ds(0, _B)], outp_hbm.at[rg, :, pl.ds(col0, _W)])

    @pl.when(cb == 0)
    def _():
        pltpu.sync_copy(dacc.at[pl.ds(0, _B * _L)], outd_hbm.at[rg])


def _combine_body(p0_ref, p1_ref, d0_ref, d1_ref, o_ref):
    num = jnp.sum(p0_ref[...], axis=0) + jnp.sum(p1_ref[...], axis=0)
    dsum = (jnp.sum(d0_ref[...], axis=(0, 2)) + jnp.sum(d1_ref[...], axis=(0, 2)))[:, None]
    o_ref[...] = num / (dsum + 1e-16)


_SPLIT = 48000


def _make_logits(row_lo, nrows, d, h, r):
    blk0 = row_lo // r
    return pl.pallas_call(
        _logits_body,
        grid=(nrows // r,),
        in_specs=[
            pl.BlockSpec((r, d), lambda i: (i + blk0, 0)),
            pl.BlockSpec((d, h), lambda i: (0, 0)),
            pl.BlockSpec((1, h), lambda i: (0, 0)),
            pl.BlockSpec((1, h), lambda i: (0, 0)),
            pl.BlockSpec((1, 1), lambda i: (0, 0)),
        ],
        out_specs=pl.BlockSpec((r, 1), lambda i: (i, 0)),
        out_shape=jax.ShapeDtypeStruct((nrows, 1), jnp.float32),
    )


def _make_pool(row_lo, nrows, d):
    import functools
    mesh = plsc.VectorSubcoreMesh(
        core_axis_name="c", subcore_axis_name="s", num_cores=_NC, num_subcores=_NS
    )
    return pl.kernel(
        functools.partial(_pool_body, row_lo, nrows),
        out_type=[
            jax.ShapeDtypeStruct((_RG, _B, d), jnp.float32),
            jax.ShapeDtypeStruct((_RG, _B * _L), jnp.float32),
        ],
        mesh=mesh,
        scratch_types=[
            pltpu.VMEM((_C, _W), jnp.float32),
            pltpu.VMEM((_C,), jnp.float32),
            pltpu.VMEM((_C,), jnp.int32),
            pltpu.VMEM((_C, _W), jnp.float32),
            pltpu.VMEM((_C,), jnp.float32),
            pltpu.VMEM((_C,), jnp.int32),
            pltpu.VMEM((_B + 1, _W), jnp.float32),
            pltpu.VMEM(((_B + 1) * _L,), jnp.float32),
            pltpu.VMEM((_W,), jnp.float32),
            pltpu.VMEM((_L,), jnp.float32),
            pltpu.SMEM((1,), jnp.int32),
            pltpu.SemaphoreType.DMA,
            pltpu.SemaphoreType.DMA,
        ],
    )


def kernel(x, batch, W1, b1, W2, b2):
    n, d = x.shape
    h = W1.shape[1]
    r = 2000
    wargs = (x, W1, b1.reshape(1, h), W2.reshape(1, h), b2.reshape(1, 1))

    parts = ((0, _SPLIT), (_SPLIT, n - _SPLIT))
    outs = []
    for row_lo, nrows in parts:
        e = _make_logits(row_lo, nrows, d, h, r)(*wargs)
        outs.append(_make_pool(row_lo, nrows, d)(x, e.reshape(nrows), batch))

    (p0, d0), (p1, d1) = outs
    out = pl.pallas_call(
        _combine_body,
        out_shape=jax.ShapeDtypeStruct((_B, d), jnp.float32),
    )(p0, p1, d0.reshape(_RG, _B, _L), d1.reshape(_RG, _B, _L))
    return out


# speedup vs baseline: 8.5088x; 1.2318x over previous
import functools

import jax
import jax.numpy as jnp
from jax import lax
from jax.experimental import pallas as pl
from jax.experimental.pallas import tpu as pltpu
from jax.experimental.pallas import tpu_sc as plsc

_B = 512
_NC = 2
_NS = 16
_NW = _NC * _NS
_CB = 4
_RG = _NW // _CB
_W = 128
_L = 16
_NV = _W // _L
_GC = 40
_CT = 160
_TX = 32
_P1 = 49920
_HL = 524


def _logits_g_body(x_ref, w1_ref, b1_ref, w2_ref, b2_ref, e_ref, g_ref):
    xb = x_ref[...]
    t = jnp.tanh(
        jnp.dot(xb, w1_ref[...], preferred_element_type=jnp.float32)
        + b1_ref[...]
    )
    logit = jnp.sum(t * w2_ref[...], axis=1, keepdims=True) + b2_ref[...]
    ev = jnp.exp(logit)
    e_ref[...] = ev
    y = xb * ev
    r = y.shape[0]
    gi = lax.broadcasted_iota(jnp.int32, (r // _L, r), 0)
    ri = lax.broadcasted_iota(jnp.int32, (r // _L, r), 1)
    sel = (ri // _L == gi).astype(jnp.float32)
    g_ref[...] = jnp.dot(sel, y, preferred_element_type=jnp.float32)


def _logits_body(x_ref, w1_ref, b1_ref, w2_ref, b2_ref, e_ref):
    t = jnp.tanh(
        jnp.dot(x_ref[...], w1_ref[...], preferred_element_type=jnp.float32)
        + b1_ref[...]
    )
    logit = jnp.sum(t * w2_ref[...], axis=1, keepdims=True) + b2_ref[...]
    e_ref[...] = jnp.exp(logit)


def _tree_sum(vals):
    while len(vals) > 1:
        nxt = [vals[i] + vals[i + 1] for i in range(0, len(vals) - 1, 2)]
        if len(vals) % 2:
            nxt.append(vals[-1])
        vals = nxt
    return vals[0]


def _pool_body(
    row_lo, n_g, has_tail,
    x_hbm, e_hbm, et_hbm, g_hbm, idx_hbm, outp_hbm, outd_hbm,
    gb0, eb0, ib0, gb1, eb1, ib1,
    txb, teb, tib,
    hxb0, heb0, hib0, hxb1, heb1, hib1,
    acc, dacc, curbuf, curden, segref, hlv,
    sem0, sem1, semt, hsem0, hsem1,
):

    cid = lax.axis_index("c")
    sid = lax.axis_index("s")
    wid = sid * _NC + cid
    cb = wid % _CB
    rg = wid // _CB
    col0 = cb * _W
    g0 = row_lo // _L

    zero = jnp.zeros((_L,), jnp.float32)

    def _zr(r, carry):
        for j in range(_NV):
            acc[r, pl.ds(j * _L, _L)] = zero
        dacc[pl.ds(r * _L, _L)] = zero
        return carry

    lax.fori_loop(0, _B + 1, _zr, 0)
    for j in range(_NV):
        curbuf[pl.ds(j * _L, _L)] = zero
    curden[...] = zero
    segref[0] = jnp.int32(_B)
    segref[1] = jnp.int32(0)

    def _flush(seg):
        for j in range(_NV):
            sl = pl.ds(j * _L, _L)
            plsc.addupdate(acc.at[seg, sl], curbuf[sl])
            curbuf[sl] = zero
        plsc.addupdate(dacc.at[pl.ds(seg * _L, _L)], curden[...])
        curden[...] = zero

    n_chunks = n_g // _GC
    base_ct = n_chunks // _RG
    n_t = base_ct + jnp.where(rg < n_chunks - base_ct * _RG, 1, 0)

    def _gstart(buf, t):
        gbuf, ebuf, ibuf, sem = buf
        c = rg + t * _RG
        pltpu.make_async_copy(
            g_hbm.at[pl.ds(c * _GC, _GC), pl.ds(col0, _W)], gbuf, sem
        ).start()
        pltpu.make_async_copy(
            e_hbm.at[pl.ds(c * _GC * _L, _GC * _L)], ebuf, sem
        ).start()
        pltpu.make_async_copy(
            idx_hbm.at[pl.ds(row_lo + c * _GC * _L, _GC * _L)], ibuf, sem
        ).start()

    def _gwait(buf):
        gbuf, ebuf, ibuf, sem = buf
        pltpu.make_async_copy(
            g_hbm.at[pl.ds(0, _GC), pl.ds(0, _W)], gbuf, sem
        ).wait()
        pltpu.make_async_copy(e_hbm.at[pl.ds(0, _GC * _L)], ebuf, sem).wait()
        pltpu.make_async_copy(idx_hbm.at[pl.ds(0, _GC * _L)], ibuf, sem).wait()

    def _gproc(buf, t):
        gbuf, ebuf, ibuf, _ = buf
        gbase = g0 + (rg + t * _RG) * _GC

        def _grp(g, carry):
            cur_seg = segref[0]
            rbase = g * _L
            e16 = ebuf[pl.ds(rbase, _L)]
            s16 = ibuf[pl.ds(rbase, _L)]
            first = s16[0]
            last = s16[_L - 1]
            hetero = first != last

            @pl.when(hetero | (first != cur_seg))
            def _():
                _flush(cur_seg)

            @pl.when(jnp.logical_not(hetero))
            def _():
                for j in range(_NV):
                    sl = pl.ds(j * _L, _L)
                    plsc.addupdate(curbuf.at[sl], gbuf[g, sl])
                plsc.addupdate(curden.at[pl.ds(0, _L)], e16)

            @pl.when(hetero)
            def _():
                cnt = segref[1]
                hlv[pl.ds(cnt * _L, _L)] = jnp.full((_L,), gbase + g, jnp.int32)
                segref[1] = cnt + 1

            segref[0] = last
            return carry

        lax.fori_loop(0, _GC, _grp, 0)

    _gstart((gb0, eb0, ib0, sem0), 0)
    bufs = ((gb0, eb0, ib0, sem0), (gb1, eb1, ib1, sem1))

    def _chunk2(t2, carry):
        for p in range(2):
            t = t2 * 2 + p

            @pl.when(t + 1 < n_t)
            def _():
                _gstart(bufs[1 - p], t + 1)

            @pl.when(t < n_t)
            def _():
                _gwait(bufs[p])
                _gproc(bufs[p], t)
        return carry

    lax.fori_loop(0, (base_ct + 2) // 2, _chunk2, 0)

    if has_tail:
        tail_r0 = row_lo + n_g * _L

        @pl.when(rg == 7)
        def _():
            pltpu.make_async_copy(et_hbm.at[pl.ds(0, _CT)], teb, semt).start()
            pltpu.make_async_copy(
                idx_hbm.at[pl.ds(tail_r0, _CT)], tib, semt
            ).start()
            pltpu.make_async_copy(et_hbm.at[pl.ds(0, _CT)], teb, semt).wait()
            pltpu.make_async_copy(idx_hbm.at[pl.ds(0, _CT)], tib, semt).wait()

            for c2 in range(_CT // _TX):
                pltpu.make_async_copy(
                    x_hbm.at[pl.ds(tail_r0 + c2 * _TX, _TX), pl.ds(col0, _W)],
                    txb, semt,
                ).start()
                pltpu.make_async_copy(
                    x_hbm.at[pl.ds(0, _TX), pl.ds(0, _W)], txb, semt
                ).wait()

                def _tgrp(g, carry):
                    cur_seg = segref[0]
                    rbase = g * _L
                    ebase = c2 * _TX + g * _L
                    e16 = teb[pl.ds(ebase, _L)]
                    s16 = tib[pl.ds(ebase, _L)]
                    first = s16[0]
                    last = s16[_L - 1]
                    hetero = first != last

                    @pl.when(hetero | (first != cur_seg))
                    def _():
                        _flush(cur_seg)

                    @pl.when(jnp.logical_not(hetero))
                    def _():
                        ev = [jnp.full((_L,), e16[k], jnp.float32)
                              for k in range(_L)]
                        for j in range(_NV):
                            sl = pl.ds(j * _L, _L)
                            contrib = _tree_sum(
                                [txb[rbase + k, sl] * ev[k] for k in range(_L)]
                            )
                            plsc.addupdate(curbuf.at[sl], contrib)
                        plsc.addupdate(curden.at[pl.ds(0, _L)], e16)

                    @pl.when(hetero)
                    def _():
                        for k in range(_L):
                            ev = jnp.full((_L,), e16[k], jnp.float32)
                            for j in range(_NV):
                                sl = pl.ds(j * _L, _L)
                                plsc.addupdate(
                                    acc.at[s16[k], sl], txb[rbase + k, sl] * ev
                                )
                            plsc.addupdate(
                                dacc.at[pl.ds(s16[k] * _L, _L)], ev * (1.0 / _L)
                            )

                    segref[0] = last
                    return carry

                lax.fori_loop(0, _TX // _L, _tgrp, 0)

    _flush(segref[0])

    cnt = segref[1]
    hbufs = ((hxb0, heb0, hib0, hsem0), (hxb1, heb1, hib1, hsem1))

    def _hstart(buf, i):
        hxb, heb, hib, sem = buf
        gid = hlv[pl.ds(i * _L, _L)][0]
        r0 = gid * _L
        pltpu.make_async_copy(
            x_hbm.at[pl.ds(r0, _L), pl.ds(col0, _W)], hxb, sem
        ).start()
        pltpu.make_async_copy(e_hbm.at[pl.ds(r0 - row_lo, _L)], heb, sem).start()
        pltpu.make_async_copy(idx_hbm.at[pl.ds(r0, _L)], hib, sem).start()

    def _hproc(buf):
        hxb, heb, hib, sem = buf
        pltpu.make_async_copy(
            x_hbm.at[pl.ds(0, _L), pl.ds(0, _W)], hxb, sem
        ).wait()
        pltpu.make_async_copy(e_hbm.at[pl.ds(0, _L)], heb, sem).wait()
        pltpu.make_async_copy(idx_hbm.at[pl.ds(0, _L)], hib, sem).wait()
        e16 = heb[...]
        s16 = hib[...]
        for k in range(_L):
            ev = jnp.full((_L,), e16[k], jnp.float32)
            for j in range(_NV):
                sl = pl.ds(j * _L, _L)
                plsc.addupdate(acc.at[s16[k], sl], hxb[k, sl] * ev)
            plsc.addupdate(dacc.at[pl.ds(s16[k] * _L, _L)], ev * (1.0 / _L))

    @pl.when(cnt > 0)
    def _():
        _hstart(hbufs[0], 0)

    def _hloop(i2, carry):
        for p in range(2):
            i = i2 * 2 + p

            @pl.when(i + 1 < cnt)
            def _():
                _hstart(hbufs[1 - p], i + 1)

            @pl.when(i < cnt)
            def _():
                _hproc(hbufs[p])
        return carry

    lax.fori_loop(0, (cnt + 1) // 2, _hloop, 0)

    pltpu.sync_copy(acc.at[pl.ds(0, _B)], outp_hbm.at[rg, :, pl.ds(col0, _W)])

    @pl.when(cb == 0)
    def _():
        pltpu.sync_copy(dacc.at[pl.ds(0, _B * _L)], outd_hbm.at[rg])


def _combine_body(*refs):
    nparts = (len(refs) - 1) // 2
    p_refs = refs[:nparts]
    d_refs = refs[nparts:-1]
    o_ref = refs[-1]
    num = _tree_sum([jnp.sum(p[...], axis=0) for p in p_refs])
    dsum = _tree_sum([jnp.sum(dr[...], axis=(0, 2)) for dr in d_refs])[:, None]
    o_ref[...] = num / (dsum + 1e-16)


def _make_logits_g(row_lo, nrows, d, h, r):
    blk0 = row_lo // r
    return pl.pallas_call(
        _logits_g_body,
        grid=(nrows // r,),
        in_specs=[
            pl.BlockSpec((r, d), lambda i: (i + blk0, 0)),
            pl.BlockSpec((d, h), lambda i: (0, 0)),
            pl.BlockSpec((1, h), lambda i: (0, 0)),
            pl.BlockSpec((1, h), lambda i: (0, 0)),
            pl.BlockSpec((1, 1), lambda i: (0, 0)),
        ],
        out_specs=[
            pl.BlockSpec((r, 1), lambda i: (i, 0)),
            pl.BlockSpec((r // _L, d), lambda i: (i, 0)),
        ],
        out_shape=[
            jax.ShapeDtypeStruct((nrows, 1), jnp.float32),
            jax.ShapeDtypeStruct((nrows // _L, d), jnp.float32),
        ],
    )


def _make_logits_e(row_lo, nrows, d, h, r):
    blk0 = row_lo // r
    return pl.pallas_call(
        _logits_body,
        grid=(nrows // r,),
        in_specs=[
            pl.BlockSpec((r, d), lambda i: (i + blk0, 0)),
            pl.BlockSpec((d, h), lambda i: (0, 0)),
            pl.BlockSpec((1, h), lambda i: (0, 0)),
            pl.BlockSpec((1, h), lambda i: (0, 0)),
            pl.BlockSpec((1, 1), lambda i: (0, 0)),
        ],
        out_specs=pl.BlockSpec((r, 1), lambda i: (i, 0)),
        out_shape=jax.ShapeDtypeStruct((nrows, 1), jnp.float32),
    )


def _make_pool(row_lo, n_g, has_tail, d):
    mesh = plsc.VectorSubcoreMesh(
        core_axis_name="c", subcore_axis_name="s", num_cores=_NC, num_subcores=_NS
    )
    return pl.kernel(
        functools.partial(_pool_body, row_lo, n_g, has_tail),
        out_type=[
            jax.ShapeDtypeStruct((_RG, _B, d), jnp.float32),
            jax.ShapeDtypeStruct((_RG, _B * _L), jnp.float32),
        ],
        mesh=mesh,
        scratch_types=[
            pltpu.VMEM((_GC, _W), jnp.float32),
            pltpu.VMEM((_GC * _L,), jnp.float32),
            pltpu.VMEM((_GC * _L,), jnp.int32),
            pltpu.VMEM((_GC, _W), jnp.float32),
            pltpu.VMEM((_GC * _L,), jnp.float32),
            pltpu.VMEM((_GC * _L,), jnp.int32),
            pltpu.VMEM((_TX, _W), jnp.float32),
            pltpu.VMEM((_CT,), jnp.float32),
            pltpu.VMEM((_CT,), jnp.int32),
            pltpu.VMEM((_L, _W), jnp.float32),
            pltpu.VMEM((_L,), jnp.float32),
            pltpu.VMEM((_L,), jnp.int32),
            pltpu.VMEM((_L, _W), jnp.float32),
            pltpu.VMEM((_L,), jnp.float32),
            pltpu.VMEM((_L,), jnp.int32),
            pltpu.VMEM((_B + 1, _W), jnp.float32),
            pltpu.VMEM(((_B + 1) * _L,), jnp.float32),
            pltpu.VMEM((_W,), jnp.float32),
            pltpu.VMEM((_L,), jnp.float32),
            pltpu.SMEM((2,), jnp.int32),
            pltpu.VMEM((_HL * _L,), jnp.int32),
            pltpu.SemaphoreType.DMA,
            pltpu.SemaphoreType.DMA,
            pltpu.SemaphoreType.DMA,
            pltpu.SemaphoreType.DMA,
            pltpu.SemaphoreType.DMA,
        ],
    )


def kernel(x, batch, W1, b1, W2, b2):
    n, d = x.shape
    h = W1.shape[1]
    r = 1920
    wargs = (x, W1, b1.reshape(1, h), W2.reshape(1, h), b2.reshape(1, 1))

    tail_r0 = 2 * _P1
    et = _make_logits_e(tail_r0, n - tail_r0, d, h, n - tail_r0)(*wargs)

    outs = []
    for pi, row_lo in enumerate((0, _P1)):
        e, g = _make_logits_g(row_lo, _P1, d, h, r)(*wargs)
        pool = _make_pool(row_lo, _P1 // _L, pi == 1, d)
        outs.append(
            pool(x, e.reshape(_P1), et.reshape(n - tail_r0), g, batch)
        )

    ps = [o[0] for o in outs]
    ds = [o[1].reshape(_RG, _B, _L) for o in outs]
    out = pl.pallas_call(
        _combine_body,
        out_shape=jax.ShapeDtypeStruct((_B, d), jnp.float32),
    )(*ps, *ds)
    return out
